# Initial kernel scaffold; baseline (speedup 1.0000x reference)
import jax
import jax.numpy as jnp
from jax.experimental import pallas as pl


def kernel(embed, node_feat, head_neighbors, tail_neighbors, head, tail, W1, b1, W2, b2, W_emb):
    raise NotImplementedError("write your pallas kernel here")


# speedup vs baseline: 2.6774x; 2.6774x over previous
import functools

import jax
import jax.numpy as jnp
from jax import lax
from jax.experimental import pallas as pl
from jax.experimental.pallas import tpu as pltpu
from jax.experimental.pallas import tpu_sc as plsc

N = 100000
D = 128
H = 64
M = 65536
K = 3
BN = 2048
NP = 100352
NT = NP // BN
NEG = float("-inf")


_PER_W = NP // 16


def _hist_body(hn_hbm, tn_hbm, out_hbm, idxh_v, idxt_v, ones_v, zbuf_v,
               hist_h_sh, hist_t_sh, sem):
  cid = lax.axis_index("c")
  sid = lax.axis_index("s")
  w = cid * 16 + sid

  for v in range(8):
    ones_v[pl.ds(v * 16, 16)] = jnp.ones((16,), jnp.float32)
  for v in range(128):
    zbuf_v[pl.ds(v * 16, 16)] = jnp.zeros((16,), jnp.float32)

  zb = sid * _PER_W
  zcopies = []
  for hist_sh in (hist_h_sh, hist_t_sh):
    for kchunk in range(3):
      zcopies.append(pltpu.async_copy(
          zbuf_v, hist_sh.at[pl.ds(zb + kchunk * 2048, 2048)], sem))
    zcopies.append(pltpu.async_copy(
        zbuf_v.at[pl.ds(0, 128)], hist_sh.at[pl.ds(zb + 3 * 2048, 128)], sem))
  pltpu.sync_copy(hn_hbm.at[pl.ds(w * 16, 16)], idxh_v)
  pltpu.sync_copy(tn_hbm.at[pl.ds(w * 16, 16)], idxt_v)
  for c in zcopies:
    c.wait()
  plsc.subcore_barrier()

  for idx_v, hist_sh in ((idxh_v, hist_h_sh), (idxt_v, hist_t_sh)):
    copies = [pltpu.async_copy(ones_v, hist_sh.at[idx_v.at[j]], sem, add=True)
              for j in range(16)]
    for c in copies:
      c.wait()
  plsc.subcore_barrier()

  c0 = pltpu.async_copy(hist_h_sh.at[pl.ds(zb, _PER_W)],
                        out_hbm.at[cid, 0, pl.ds(zb, _PER_W)], sem)
  c1 = pltpu.async_copy(hist_t_sh.at[pl.ds(zb, _PER_W)],
                        out_hbm.at[cid, 1, pl.ds(zb, _PER_W)], sem)
  c0.wait()
  c1.wait()


@functools.cache
def _hist_kernel_fn():
  return pl.kernel(
      _hist_body,
      out_type=jax.ShapeDtypeStruct((2, 2, NP), jnp.float32),
      mesh=plsc.VectorSubcoreMesh(core_axis_name="c", subcore_axis_name="s",
                                  num_cores=2, num_subcores=16),
      scratch_types=[
          pltpu.VMEM((16, 128), jnp.int32),
          pltpu.VMEM((16, 128), jnp.int32),
          pltpu.VMEM((128,), jnp.float32),
          pltpu.VMEM((2048,), jnp.float32),
          pltpu.VMEM_SHARED((NP,), jnp.float32),
          pltpu.VMEM_SHARED((NP,), jnp.float32),
          pltpu.SemaphoreType.DMA,
      ],
  )


def _hist_kernel(hn2, tn2):
  return _hist_kernel_fn()(hn2, tn2)


def _score_body(emb_ref, w1a_ref, w1b_ref, b1_ref, w2_ref, cen_ref, emb_hbm,
                sh_ref, st_ref, anc_v, cvec_v, sem):
  i = pl.program_id(0)

  @pl.when(i == 0)
  def _init():
    ch = pltpu.make_async_copy(emb_hbm.at[pl.ds(cen_ref[0], 1)],
                               anc_v.at[pl.ds(0, 1)], sem)
    ct = pltpu.make_async_copy(emb_hbm.at[pl.ds(cen_ref[1], 1)],
                               anc_v.at[pl.ds(1, 1)], sem)
    ch.start()
    ct.start()
    ch.wait()
    ct.wait()
    cvec_v[...] = (
        jnp.dot(anc_v[...], w1a_ref[...], preferred_element_type=jnp.float32)
        + b1_ref[...])

  x = jnp.dot(emb_ref[...], w1b_ref[...], preferred_element_type=jnp.float32)
  hh = jnp.maximum(x + cvec_v[0:1, :], 0.0)
  ht = jnp.maximum(x + cvec_v[1:2, :], 0.0)
  sh_ref[...] = jnp.dot(hh, w2_ref[...], preferred_element_type=jnp.float32)
  st_ref[...] = jnp.dot(ht, w2_ref[...], preferred_element_type=jnp.float32)


def _scores(embed, W1a, W1b, b1, W2, centers):
  return pl.pallas_call(
      _score_body,
      grid=(NT,),
      in_specs=[
          pl.BlockSpec((BN, D), lambda i: (i, 0)),
          pl.BlockSpec((D, H), lambda i: (0, 0)),
          pl.BlockSpec((D, H), lambda i: (0, 0)),
          pl.BlockSpec((1, H), lambda i: (0, 0)),
          pl.BlockSpec((H, 1), lambda i: (0, 0)),
          pl.BlockSpec(memory_space=pltpu.SMEM),
          pl.BlockSpec(memory_space=pltpu.HBM),
      ],
      out_specs=[
          pl.BlockSpec((BN, 1), lambda i: (i, 0)),
          pl.BlockSpec((BN, 1), lambda i: (i, 0)),
      ],
      out_shape=[
          jax.ShapeDtypeStruct((NP, 1), jnp.float32),
          jax.ShapeDtypeStruct((NP, 1), jnp.float32),
      ],
      scratch_shapes=[
          pltpu.VMEM((2, D), jnp.float32),
          pltpu.VMEM((2, H), jnp.float32),
          pltpu.SemaphoreType.DMA,
      ],
  )(embed, W1a, W1b, b1, W2, centers, embed)


def _wsum_body(nf_ref, h00_ref, h01_ref, h10_ref, h11_ref, out_ref):
  i = pl.program_id(0)
  mh = h00_ref[...] + h10_ref[...]
  mt = h01_ref[...] + h11_ref[...]
  rowid = i * BN + lax.broadcasted_iota(jnp.int32, (BN, 1), 0)
  blk = jnp.where(rowid < N, nf_ref[...], 0.0)
  ch = lax.dot_general(mh, blk, (((0,), (0,)), ((), ())),
                       preferred_element_type=jnp.float32)
  ct = lax.dot_general(mt, blk, (((0,), (0,)), ((), ())),
                       preferred_element_type=jnp.float32)
  contrib = jnp.concatenate([ch, ct], axis=0)

  @pl.when(i == 0)
  def _init():
    out_ref[...] = jnp.zeros_like(out_ref)

  out_ref[...] += contrib


def _wsum(node_feat, h00, h01, h10, h11):
  hspec = pl.BlockSpec((BN, 1), lambda i: (i, 0))
  return pl.pallas_call(
      _wsum_body,
      grid=(NT,),
      in_specs=[
          pl.BlockSpec((BN, D), lambda i: (i, 0)),
          hspec, hspec, hspec, hspec,
      ],
      out_specs=pl.BlockSpec((2, D), lambda i: (0, 0)),
      out_shape=jax.ShapeDtypeStruct((2, D), jnp.float32),
  )(node_feat, h00, h01, h10, h11)


def _final_body(sh_ref, st_ref, h00_ref, h01_ref, h10_ref, h11_ref,
                wsum_ref, wemb_ref, cen_ref, nf_hbm, out_ref, rows_v, sem):
  lin = (lax.broadcasted_iota(jnp.int32, (NP // 128, 128), 0) * 128
         + lax.broadcasted_iota(jnp.int32, (NP // 128, 128), 1))
  big = jnp.int32(2 ** 30)

  mult_h = h00_ref[...] + h10_ref[...]
  mult_t = h01_ref[...] + h11_ref[...]

  copies = []
  sels = []
  uses = []
  for a, (sc_ref, mult) in enumerate(
      ((sh_ref, mult_h), (st_ref, mult_t))):
    s_w = jnp.where(mult > 0.0, sc_ref[...], NEG)
    rem = jnp.float32(K)
    for k in range(K):
      m = jnp.max(s_w)
      idx = jnp.min(jnp.where(s_w == m, lin, big))
      cnt = jnp.sum(jnp.where(lin == idx, mult, 0.0))
      sels.append(idx)
      uses.append((rem > 0.0).astype(jnp.float32))
      rem = rem - cnt
      s_w = jnp.where(lin == idx, NEG, s_w)
      cp = pltpu.make_async_copy(nf_hbm.at[pl.ds(idx, 1)],
                                 rows_v.at[pl.ds(a * K + k, 1)], sem)
      cp.start()
      copies.append(cp)

  for a in range(2):
    cp = pltpu.make_async_copy(nf_hbm.at[pl.ds(cen_ref[a], 1)],
                               rows_v.at[pl.ds(2 * K + a, 1)], sem)
    cp.start()
    copies.append(cp)
  for cp in copies:
    cp.wait()

  eff = []
  for j in range(2 * K):
    d = uses[j]
    for i in range(j):
      d = d * (1.0 - uses[i] * (sels[i] == sels[j]).astype(jnp.float32))
    eff.append(d)

  vrows = []
  for a, mult in enumerate((mult_h, mult_t)):
    corr = jnp.zeros((1, D), jnp.float32)
    for j in range(2 * K):
      cnt_aj = jnp.sum(jnp.where(lin == sels[j], mult, 0.0))
      corr = corr + (eff[j] * cnt_aj) * rows_v[j:j + 1, :]
    keep = jnp.float32(1.0)
    for j in range(2 * K):
      hit = uses[j] * (sels[j] == cen_ref[a]).astype(jnp.float32)
      keep = keep * (1.0 - hit)
    agg = (wsum_ref[a:a + 1, :] - corr) * jnp.float32(1.0 / M)
    vrows.append(keep * rows_v[2 * K + a:2 * K + a + 1, :] + agg)

  v = jnp.concatenate(vrows, axis=0)
  out_ref[...] = jnp.dot(v, wemb_ref[...], preferred_element_type=jnp.float32)


def _final(sh, st, h00, h01, h10, h11, wsum, W_emb, centers, node_feat):
  full = pl.BlockSpec((NP // 128, 128), lambda: (0, 0))
  return pl.pallas_call(
      _final_body,
      in_specs=[
          full, full, full, full, full, full,
          pl.BlockSpec((2, D), lambda: (0, 0)),
          pl.BlockSpec((D, D), lambda: (0, 0)),
          pl.BlockSpec(memory_space=pltpu.SMEM),
          pl.BlockSpec(memory_space=pltpu.HBM),
      ],
      out_specs=pl.BlockSpec((2, D), lambda: (0, 0)),
      out_shape=jax.ShapeDtypeStruct((2, D), jnp.float32),
      scratch_shapes=[
          pltpu.VMEM((2 * K + 2, D), jnp.float32),
          pltpu.SemaphoreType.DMA,
      ],
  )(sh, st, h00, h01, h10, h11, wsum, W_emb, centers, node_feat)


def kernel(embed, node_feat, head_neighbors, tail_neighbors, head, tail,
           W1, b1, W2, b2, W_emb):
  centers = jnp.stack([head, tail]).astype(jnp.int32)
  hn2 = head_neighbors.reshape(M // 128, 128)
  tn2 = tail_neighbors.reshape(M // 128, 128)

  hist = _hist_kernel(hn2, tn2)
  h00 = hist[0, 0].reshape(NP, 1)
  h01 = hist[0, 1].reshape(NP, 1)
  h10 = hist[1, 0].reshape(NP, 1)
  h11 = hist[1, 1].reshape(NP, 1)

  W1a = W1[:D]
  W1b = W1[D:]
  sh, st = _scores(embed, W1a, W1b, b1.reshape(1, H), W2, centers)
  wsum = _wsum(node_feat, h00, h01, h10, h11)

  out = _final(sh.reshape(NP // 128, 128), st.reshape(NP // 128, 128),
               h00.reshape(NP // 128, 128), h01.reshape(NP // 128, 128),
               h10.reshape(NP // 128, 128), h11.reshape(NP // 128, 128),
               wsum, W_emb, centers, node_feat)
  return out


# speedup vs baseline: 6.9954x; 2.6127x over previous
import functools

import jax
import jax.numpy as jnp
from jax import lax
from jax.experimental import pallas as pl
from jax.experimental.pallas import tpu as pltpu
from jax.experimental.pallas import tpu_sc as plsc

N = 100000
D = 128
H = 64
M = 65536
K = 3
BN = 2048
NP = 100352
NT = NP // BN
NEG = float("-inf")


_PER_W = NP // 16


def _hist_body(hn_hbm, tn_hbm, out_hbm, idxh_v, idxt_v, ones_v, zbuf_v,
               hist_h_sh, hist_t_sh, sem):
  cid = lax.axis_index("c")
  sid = lax.axis_index("s")
  w = cid * 16 + sid

  for v in range(8):
    ones_v[pl.ds(v * 16, 16)] = jnp.ones((16,), jnp.float32)
  for v in range(128):
    zbuf_v[pl.ds(v * 16, 16)] = jnp.zeros((16,), jnp.float32)

  zb = sid * _PER_W
  zcopies = []
  for hist_sh in (hist_h_sh, hist_t_sh):
    for kchunk in range(3):
      zcopies.append(pltpu.async_copy(
          zbuf_v, hist_sh.at[pl.ds(zb + kchunk * 2048, 2048)], sem))
    zcopies.append(pltpu.async_copy(
        zbuf_v.at[pl.ds(0, 128)], hist_sh.at[pl.ds(zb + 3 * 2048, 128)], sem))
  pltpu.sync_copy(hn_hbm.at[pl.ds(w * 16, 16)], idxh_v)
  pltpu.sync_copy(tn_hbm.at[pl.ds(w * 16, 16)], idxt_v)
  for c in zcopies:
    c.wait()
  plsc.subcore_barrier()

  for idx_v, hist_sh in ((idxh_v, hist_h_sh), (idxt_v, hist_t_sh)):
    copies = [pltpu.async_copy(ones_v, hist_sh.at[idx_v.at[j]], sem, add=True)
              for j in range(16)]
    for c in copies:
      c.wait()
  plsc.subcore_barrier()

  c0 = pltpu.async_copy(hist_h_sh.at[pl.ds(zb, _PER_W)],
                        out_hbm.at[cid, 0, pl.ds(zb, _PER_W)], sem)
  c1 = pltpu.async_copy(hist_t_sh.at[pl.ds(zb, _PER_W)],
                        out_hbm.at[cid, 1, pl.ds(zb, _PER_W)], sem)
  c0.wait()
  c1.wait()


@functools.cache
def _hist_kernel_fn():
  return pl.kernel(
      _hist_body,
      out_type=jax.ShapeDtypeStruct((2, 2, NP), jnp.float32),
      mesh=plsc.VectorSubcoreMesh(core_axis_name="c", subcore_axis_name="s",
                                  num_cores=2, num_subcores=16),
      scratch_types=[
          pltpu.VMEM((16, 128), jnp.int32),
          pltpu.VMEM((16, 128), jnp.int32),
          pltpu.VMEM((128,), jnp.float32),
          pltpu.VMEM((2048,), jnp.float32),
          pltpu.VMEM_SHARED((NP,), jnp.float32),
          pltpu.VMEM_SHARED((NP,), jnp.float32),
          pltpu.SemaphoreType.DMA,
      ],
  )


def _hist_kernel(hn2, tn2):
  return _hist_kernel_fn()(hn2, tn2)


def _score_body(emb_ref, w1a_ref, w1b_ref, b1_ref, w2_ref, cen_ref, emb_hbm,
                sh_ref, st_ref, anc_v, cvec_v, sem):
  i = pl.program_id(0)

  @pl.when(i == 0)
  def _init():
    ch = pltpu.make_async_copy(emb_hbm.at[pl.ds(cen_ref[0], 1)],
                               anc_v.at[pl.ds(0, 1)], sem)
    ct = pltpu.make_async_copy(emb_hbm.at[pl.ds(cen_ref[1], 1)],
                               anc_v.at[pl.ds(1, 1)], sem)
    ch.start()
    ct.start()
    ch.wait()
    ct.wait()
    cvec_v[...] = (
        lax.dot_general(w1a_ref[...], anc_v[...], (((0,), (1,)), ((), ())),
                        preferred_element_type=jnp.float32)
        + b1_ref[...])

  xt = lax.dot_general(w1b_ref[...], emb_ref[...], (((0,), (1,)), ((), ())),
                       preferred_element_type=jnp.float32)
  hh = jnp.maximum(xt + cvec_v[:, 0:1], 0.0)
  ht = jnp.maximum(xt + cvec_v[:, 1:2], 0.0)
  sh = lax.dot_general(w2_ref[...], hh, (((0,), (0,)), ((), ())),
                       preferred_element_type=jnp.float32)
  st = lax.dot_general(w2_ref[...], ht, (((0,), (0,)), ((), ())),
                       preferred_element_type=jnp.float32)
  sh_ref[...] = sh.reshape(1, 1, BN)
  st_ref[...] = st.reshape(1, 1, BN)


def _scores(embed, W1a, W1b, b1, W2, centers):
  return pl.pallas_call(
      _score_body,
      grid=(NT,),
      in_specs=[
          pl.BlockSpec((BN, D), lambda i: (i, 0)),
          pl.BlockSpec((D, H), lambda i: (0, 0)),
          pl.BlockSpec((D, H), lambda i: (0, 0)),
          pl.BlockSpec((H, 1), lambda i: (0, 0)),
          pl.BlockSpec((H, 1), lambda i: (0, 0)),
          pl.BlockSpec(memory_space=pltpu.SMEM),
          pl.BlockSpec(memory_space=pltpu.HBM),
      ],
      out_specs=[
          pl.BlockSpec((1, 1, BN), lambda i: (i, 0, 0)),
          pl.BlockSpec((1, 1, BN), lambda i: (i, 0, 0)),
      ],
      out_shape=[
          jax.ShapeDtypeStruct((NT, 1, BN), jnp.float32),
          jax.ShapeDtypeStruct((NT, 1, BN), jnp.float32),
      ],
      scratch_shapes=[
          pltpu.VMEM((2, D), jnp.float32),
          pltpu.VMEM((H, 2), jnp.float32),
          pltpu.SemaphoreType.DMA,
      ],
  )(embed, W1a, W1b, b1, W2, centers, embed)


def _wsum_body(nf_ref, h4_ref, out_ref):
  i = pl.program_id(0)
  h4 = h4_ref[...]
  mh = h4[0, 0] + h4[1, 0]
  mt = h4[0, 1] + h4[1, 1]
  rowid = (i * BN
           + lax.broadcasted_iota(jnp.int32, (BN, 1), 0))
  blk = jnp.where(rowid < N, nf_ref[...], 0.0)
  contrib = jnp.zeros((2, D), jnp.float32)
  for r in range(BN // 128):
    lhs = jnp.concatenate([mh[r:r + 1, :], mt[r:r + 1, :]], axis=0)
    contrib = contrib + jnp.dot(lhs, blk[r * 128:(r + 1) * 128, :],
                                preferred_element_type=jnp.float32)

  @pl.when(i == 0)
  def _init():
    out_ref[...] = jnp.zeros_like(out_ref)

  out_ref[...] += contrib


def _wsum(node_feat, hist4):
  return pl.pallas_call(
      _wsum_body,
      grid=(NT,),
      in_specs=[
          pl.BlockSpec((BN, D), lambda i: (i, 0)),
          pl.BlockSpec((2, 2, BN // 128, 128), lambda i: (0, 0, i, 0)),
      ],
      out_specs=pl.BlockSpec((2, D), lambda i: (0, 0)),
      out_shape=jax.ShapeDtypeStruct((2, D), jnp.float32),
  )(node_feat, hist4)


def _final_body(sh_ref, st_ref, h4_ref, wsum_ref, wemb_ref, cen_ref, nf_hbm,
                out_ref, rows_v, sem):
  lin = (lax.broadcasted_iota(jnp.int32, (NP // 128, 128), 0) * 128
         + lax.broadcasted_iota(jnp.int32, (NP // 128, 128), 1))
  big = jnp.int32(2 ** 30)

  h4 = h4_ref[...]
  mult_h = h4[0, 0] + h4[1, 0]
  mult_t = h4[0, 1] + h4[1, 1]

  copies = []
  sels = []
  uses = []
  for a, (sc_ref, mult) in enumerate(
      ((sh_ref, mult_h), (st_ref, mult_t))):
    s_w = jnp.where(mult > 0.0, sc_ref[...], NEG)
    rem = jnp.float32(K)
    for k in range(K):
      m = jnp.max(s_w)
      idx = jnp.min(jnp.where(s_w == m, lin, big))
      cnt = jnp.sum(jnp.where(lin == idx, mult, 0.0))
      sels.append(idx)
      uses.append((rem > 0.0).astype(jnp.float32))
      rem = rem - cnt
      s_w = jnp.where(lin == idx, NEG, s_w)
      cp = pltpu.make_async_copy(nf_hbm.at[pl.ds(idx, 1)],
                                 rows_v.at[pl.ds(a * K + k, 1)], sem)
      cp.start()
      copies.append(cp)

  for a in range(2):
    cp = pltpu.make_async_copy(nf_hbm.at[pl.ds(cen_ref[a], 1)],
                               rows_v.at[pl.ds(2 * K + a, 1)], sem)
    cp.start()
    copies.append(cp)
  for cp in copies:
    cp.wait()

  eff = []
  for j in range(2 * K):
    d = uses[j]
    for i in range(j):
      d = d * (1.0 - uses[i] * (sels[i] == sels[j]).astype(jnp.float32))
    eff.append(d)

  vrows = []
  for a, mult in enumerate((mult_h, mult_t)):
    corr = jnp.zeros((1, D), jnp.float32)
    for j in range(2 * K):
      cnt_aj = jnp.sum(jnp.where(lin == sels[j], mult, 0.0))
      corr = corr + (eff[j] * cnt_aj) * rows_v[j:j + 1, :]
    keep = jnp.float32(1.0)
    for j in range(2 * K):
      hit = uses[j] * (sels[j] == cen_ref[a]).astype(jnp.float32)
      keep = keep * (1.0 - hit)
    agg = (wsum_ref[a:a + 1, :] - corr) * jnp.float32(1.0 / M)
    vrows.append(keep * rows_v[2 * K + a:2 * K + a + 1, :] + agg)

  v = jnp.concatenate(vrows, axis=0)
  out_ref[...] = jnp.dot(v, wemb_ref[...], preferred_element_type=jnp.float32)


def _final(sh, st, hist4, wsum, W_emb, centers, node_feat):
  full = pl.BlockSpec((NP // 128, 128), lambda: (0, 0))
  return pl.pallas_call(
      _final_body,
      in_specs=[
          full, full,
          pl.BlockSpec((2, 2, NP // 128, 128), lambda: (0, 0, 0, 0)),
          pl.BlockSpec((2, D), lambda: (0, 0)),
          pl.BlockSpec((D, D), lambda: (0, 0)),
          pl.BlockSpec(memory_space=pltpu.SMEM),
          pl.BlockSpec(memory_space=pltpu.HBM),
      ],
      out_specs=pl.BlockSpec((2, D), lambda: (0, 0)),
      out_shape=jax.ShapeDtypeStruct((2, D), jnp.float32),
      scratch_shapes=[
          pltpu.VMEM((2 * K + 2, D), jnp.float32),
          pltpu.SemaphoreType.DMA,
      ],
  )(sh, st, hist4, wsum, W_emb, centers, node_feat)


def kernel(embed, node_feat, head_neighbors, tail_neighbors, head, tail,
           W1, b1, W2, b2, W_emb):
  centers = jnp.stack([head, tail]).astype(jnp.int32)
  hn2 = head_neighbors.reshape(M // 128, 128)
  tn2 = tail_neighbors.reshape(M // 128, 128)

  hist = _hist_kernel(hn2, tn2)
  hist4 = hist.reshape(2, 2, NP // 128, 128)

  W1a = W1[:D]
  W1b = W1[D:]
  sh, st = _scores(embed, W1a, W1b, b1.reshape(H, 1), W2, centers)
  wsum = _wsum(node_feat, hist4)

  out = _final(sh.reshape(NP // 128, 128), st.reshape(NP // 128, 128),
               hist4, wsum, W_emb, centers, node_feat)
  return out


# speedup vs baseline: 9.0797x; 1.2980x over previous
import functools

import jax
import jax.numpy as jnp
from jax import lax
from jax.experimental import pallas as pl
from jax.experimental.pallas import tpu as pltpu
from jax.experimental.pallas import tpu_sc as plsc

N = 100000
D = 128
H = 64
M = 65536
K = 3
BN = 4096
NP = 102400
NT = NP // BN
NEG = float("-inf")


_PER_W = NP // 16


def _hist_body(hn_hbm, tn_hbm, out_hbm, idxh_v, idxt_v, ones_v, zbuf_v,
               hist_h_sh, hist_t_sh, sem):
  cid = lax.axis_index("c")
  sid = lax.axis_index("s")
  w = cid * 16 + sid

  for v in range(8):
    ones_v[pl.ds(v * 16, 16)] = jnp.ones((16,), jnp.float32)
  for v in range(128):
    zbuf_v[pl.ds(v * 16, 16)] = jnp.zeros((16,), jnp.float32)

  zb = sid * _PER_W
  zcopies = []
  for hist_sh in (hist_h_sh, hist_t_sh):
    for kchunk in range(3):
      zcopies.append(pltpu.async_copy(
          zbuf_v, hist_sh.at[pl.ds(zb + kchunk * 2048, 2048)], sem))
    zcopies.append(pltpu.async_copy(
        zbuf_v.at[pl.ds(0, 256)], hist_sh.at[pl.ds(zb + 3 * 2048, 256)], sem))
  pltpu.sync_copy(hn_hbm.at[pl.ds(w * 16, 16)], idxh_v)
  pltpu.sync_copy(tn_hbm.at[pl.ds(w * 16, 16)], idxt_v)
  for c in zcopies:
    c.wait()
  plsc.subcore_barrier()

  for idx_v, hist_sh in ((idxh_v, hist_h_sh), (idxt_v, hist_t_sh)):
    copies = [pltpu.async_copy(ones_v, hist_sh.at[idx_v.at[j]], sem, add=True)
              for j in range(16)]
    for c in copies:
      c.wait()
  plsc.subcore_barrier()

  c0 = pltpu.async_copy(hist_h_sh.at[pl.ds(zb, _PER_W)],
                        out_hbm.at[cid, 0, pl.ds(zb, _PER_W)], sem)
  c1 = pltpu.async_copy(hist_t_sh.at[pl.ds(zb, _PER_W)],
                        out_hbm.at[cid, 1, pl.ds(zb, _PER_W)], sem)
  c0.wait()
  c1.wait()


@functools.cache
def _hist_kernel_fn():
  return pl.kernel(
      _hist_body,
      out_type=jax.ShapeDtypeStruct((2, 2, NP), jnp.float32),
      mesh=plsc.VectorSubcoreMesh(core_axis_name="c", subcore_axis_name="s",
                                  num_cores=2, num_subcores=16),
      scratch_types=[
          pltpu.VMEM((16, 128), jnp.int32),
          pltpu.VMEM((16, 128), jnp.int32),
          pltpu.VMEM((128,), jnp.float32),
          pltpu.VMEM((2048,), jnp.float32),
          pltpu.VMEM_SHARED((NP,), jnp.float32),
          pltpu.VMEM_SHARED((NP,), jnp.float32),
          pltpu.SemaphoreType.DMA,
      ],
  )


def _hist_kernel(hn2, tn2):
  return _hist_kernel_fn()(hn2, tn2)


def _score_body(emb_ref, w1a_ref, w1b_ref, b1_ref, w2_ref, cen_ref, emb_hbm,
                sh_ref, st_ref, anc_v, cvec_v, sem):
  i = pl.program_id(0)

  @pl.when(i == 0)
  def _init():
    ch = pltpu.make_async_copy(emb_hbm.at[pl.ds(cen_ref[0], 1)],
                               anc_v.at[pl.ds(0, 1)], sem)
    ct = pltpu.make_async_copy(emb_hbm.at[pl.ds(cen_ref[1], 1)],
                               anc_v.at[pl.ds(1, 1)], sem)
    ch.start()
    ct.start()
    ch.wait()
    ct.wait()
    cvec_v[...] = (
        lax.dot_general(w1a_ref[...], anc_v[...], (((0,), (1,)), ((), ())),
                        preferred_element_type=jnp.float32)
        + b1_ref[...])

  xt = lax.dot_general(w1b_ref[...], emb_ref[...], (((0,), (1,)), ((), ())),
                       preferred_element_type=jnp.float32)
  hh = jnp.maximum(xt + cvec_v[:, 0:1], 0.0)
  ht = jnp.maximum(xt + cvec_v[:, 1:2], 0.0)
  sh = lax.dot_general(w2_ref[...], hh, (((0,), (0,)), ((), ())),
                       preferred_element_type=jnp.float32)
  st = lax.dot_general(w2_ref[...], ht, (((0,), (0,)), ((), ())),
                       preferred_element_type=jnp.float32)
  sh_ref[...] = sh.reshape(1, 1, BN)
  st_ref[...] = st.reshape(1, 1, BN)


def _scores(embed, W1a, W1b, b1, W2, centers):
  return pl.pallas_call(
      _score_body,
      grid=(NT,),
      in_specs=[
          pl.BlockSpec((BN, D), lambda i: (i, 0)),
          pl.BlockSpec((D, H), lambda i: (0, 0)),
          pl.BlockSpec((D, H), lambda i: (0, 0)),
          pl.BlockSpec((H, 1), lambda i: (0, 0)),
          pl.BlockSpec((H, 1), lambda i: (0, 0)),
          pl.BlockSpec(memory_space=pltpu.SMEM),
          pl.BlockSpec(memory_space=pltpu.HBM),
      ],
      out_specs=[
          pl.BlockSpec((1, 1, BN), lambda i: (i, 0, 0)),
          pl.BlockSpec((1, 1, BN), lambda i: (i, 0, 0)),
      ],
      out_shape=[
          jax.ShapeDtypeStruct((NT, 1, BN), jnp.float32),
          jax.ShapeDtypeStruct((NT, 1, BN), jnp.float32),
      ],
      scratch_shapes=[
          pltpu.VMEM((2, D), jnp.float32),
          pltpu.VMEM((H, 2), jnp.float32),
          pltpu.SemaphoreType.DMA,
      ],
  )(embed, W1a, W1b, b1, W2, centers, embed)


def _wsum_body(nf_ref, h4_ref, out_ref):
  i = pl.program_id(0)
  h4 = h4_ref[...]
  mh = h4[0, 0] + h4[1, 0]
  mt = h4[0, 1] + h4[1, 1]
  rowid = (i * BN
           + lax.broadcasted_iota(jnp.int32, (BN, 1), 0))
  blk = jnp.where(rowid < N, nf_ref[...], 0.0)
  accs = [jnp.zeros((2, D), jnp.float32) for _ in range(8)]
  for r in range(BN // 128):
    lhs = jnp.concatenate([mh[r:r + 1, :], mt[r:r + 1, :]], axis=0)
    accs[r % 8] = accs[r % 8] + jnp.dot(lhs, blk[r * 128:(r + 1) * 128, :],
                                        preferred_element_type=jnp.float32)
  contrib = ((accs[0] + accs[1]) + (accs[2] + accs[3])) + (
      (accs[4] + accs[5]) + (accs[6] + accs[7]))

  @pl.when(i == 0)
  def _init():
    out_ref[...] = jnp.zeros_like(out_ref)

  out_ref[...] += contrib


def _wsum(node_feat, hist4):
  return pl.pallas_call(
      _wsum_body,
      grid=(NT,),
      in_specs=[
          pl.BlockSpec((BN, D), lambda i: (i, 0)),
          pl.BlockSpec((2, 2, BN // 128, 128), lambda i: (0, 0, i, 0)),
      ],
      out_specs=pl.BlockSpec((2, D), lambda i: (0, 0)),
      out_shape=jax.ShapeDtypeStruct((2, D), jnp.float32),
  )(node_feat, hist4)


def _final_body(sh_ref, st_ref, h4_ref, wsum_ref, wemb_ref, cen_ref, nf_hbm,
                out_ref, rows_v, sem):
  lin = (lax.broadcasted_iota(jnp.int32, (NP // 128, 128), 0) * 128
         + lax.broadcasted_iota(jnp.int32, (NP // 128, 128), 1))
  big = jnp.int32(2 ** 30)

  h4 = h4_ref[...]
  mult_h = h4[0, 0] + h4[1, 0]
  mult_t = h4[0, 1] + h4[1, 1]

  copies = []
  sels = []
  uses = []
  for a, (sc_ref, mult) in enumerate(
      ((sh_ref, mult_h), (st_ref, mult_t))):
    s_w = jnp.where(mult > 0.0, sc_ref[...], NEG)
    rem = jnp.float32(K)
    for k in range(K):
      m = jnp.max(s_w)
      idx = jnp.min(jnp.where(s_w == m, lin, big))
      cnt = jnp.sum(jnp.where(lin == idx, mult, 0.0))
      sels.append(idx)
      uses.append((rem > 0.0).astype(jnp.float32))
      rem = rem - cnt
      s_w = jnp.where(lin == idx, NEG, s_w)
      cp = pltpu.make_async_copy(nf_hbm.at[pl.ds(idx, 1)],
                                 rows_v.at[pl.ds(a * K + k, 1)], sem)
      cp.start()
      copies.append(cp)

  for a in range(2):
    cp = pltpu.make_async_copy(nf_hbm.at[pl.ds(cen_ref[a], 1)],
                               rows_v.at[pl.ds(2 * K + a, 1)], sem)
    cp.start()
    copies.append(cp)
  for cp in copies:
    cp.wait()

  eff = []
  for j in range(2 * K):
    d = uses[j]
    for i in range(j):
      d = d * (1.0 - uses[i] * (sels[i] == sels[j]).astype(jnp.float32))
    eff.append(d)

  vrows = []
  for a, mult in enumerate((mult_h, mult_t)):
    corr = jnp.zeros((1, D), jnp.float32)
    for j in range(2 * K):
      cnt_aj = jnp.sum(jnp.where(lin == sels[j], mult, 0.0))
      corr = corr + (eff[j] * cnt_aj) * rows_v[j:j + 1, :]
    keep = jnp.float32(1.0)
    for j in range(2 * K):
      hit = uses[j] * (sels[j] == cen_ref[a]).astype(jnp.float32)
      keep = keep * (1.0 - hit)
    agg = (wsum_ref[a:a + 1, :] - corr) * jnp.float32(1.0 / M)
    vrows.append(keep * rows_v[2 * K + a:2 * K + a + 1, :] + agg)

  v = jnp.concatenate(vrows, axis=0)
  out_ref[...] = jnp.dot(v, wemb_ref[...], preferred_element_type=jnp.float32)


def _final(sh, st, hist4, wsum, W_emb, centers, node_feat):
  full = pl.BlockSpec((NP // 128, 128), lambda: (0, 0))
  return pl.pallas_call(
      _final_body,
      in_specs=[
          full, full,
          pl.BlockSpec((2, 2, NP // 128, 128), lambda: (0, 0, 0, 0)),
          pl.BlockSpec((2, D), lambda: (0, 0)),
          pl.BlockSpec((D, D), lambda: (0, 0)),
          pl.BlockSpec(memory_space=pltpu.SMEM),
          pl.BlockSpec(memory_space=pltpu.HBM),
      ],
      out_specs=pl.BlockSpec((2, D), lambda: (0, 0)),
      out_shape=jax.ShapeDtypeStruct((2, D), jnp.float32),
      scratch_shapes=[
          pltpu.VMEM((2 * K + 2, D), jnp.float32),
          pltpu.SemaphoreType.DMA,
      ],
  )(sh, st, hist4, wsum, W_emb, centers, node_feat)


def kernel(embed, node_feat, head_neighbors, tail_neighbors, head, tail,
           W1, b1, W2, b2, W_emb):
  centers = jnp.stack([head, tail]).astype(jnp.int32)
  hn2 = head_neighbors.reshape(M // 128, 128)
  tn2 = tail_neighbors.reshape(M // 128, 128)

  hist = _hist_kernel(hn2, tn2)
  hist4 = hist.reshape(2, 2, NP // 128, 128)

  W1a = W1[:D]
  W1b = W1[D:]
  sh, st = _scores(embed, W1a, W1b, b1.reshape(H, 1), W2, centers)
  wsum = _wsum(node_feat, hist4)

  out = _final(sh.reshape(NP // 128, 128), st.reshape(NP // 128, 128),
               hist4, wsum, W_emb, centers, node_feat)
  return out
